# Initial kernel scaffold; baseline (speedup 1.0000x reference)
import jax
import jax.numpy as jnp
from jax.experimental import pallas as pl


def kernel(hidden_states, gate_w, w1s, w2s, w3s):
    raise NotImplementedError("write your pallas kernel here")


# speedup vs baseline: 2.6707x; 2.6707x over previous
import functools

import jax
import jax.numpy as jnp
from jax.experimental import pallas as pl
from jax.experimental.pallas import tpu as pltpu


_M = 512
_IC = 512


def _router_body(x_ref, g_ref, w_ref, e_ref):
    x = x_ref[...]
    logits = jnp.dot(x, g_ref[...], preferred_element_type=jnp.float32)
    t, e = logits.shape
    iota = jax.lax.broadcasted_iota(jnp.int32, (t, e), 1)
    m1 = jnp.max(logits, axis=1, keepdims=True)
    i1 = jnp.min(jnp.where(logits == m1, iota, e), axis=1, keepdims=True)
    masked = jnp.where(iota == i1, -jnp.inf, logits)
    m2 = jnp.max(masked, axis=1, keepdims=True)
    i2 = jnp.min(jnp.where(masked == m2, iota, e), axis=1, keepdims=True)
    wa = jax.nn.sigmoid(m1 - m2)
    w_ref[:, 0:1] = wa
    w_ref[:, 1:2] = 1.0 - wa
    e_ref[:, 0:1] = i1
    e_ref[:, 1:2] = i2


def _mlp_body(meta_ref, x_ref, rw_ref, w1_ref, w3_ref, w2_ref, out_ref):
    p = pl.program_id(0)
    ic = pl.program_id(1)
    lo = meta_ref[p, 2]
    hi = meta_ref[p, 3]
    init = meta_ref[p, 4]

    x = x_ref[...]
    g = jnp.dot(x, w1_ref[0], preferred_element_type=jnp.float32)
    u = jnp.dot(x, w3_ref[0], preferred_element_type=jnp.float32)
    h = (g * jax.nn.sigmoid(g)) * u
    rows = jax.lax.broadcasted_iota(jnp.int32, (x.shape[0], 1), 0)
    keep = (rows >= lo) & (rows < hi)
    h = jnp.where(keep, h * rw_ref[...], 0.0)
    contrib = jnp.dot(h, w2_ref[0], preferred_element_type=jnp.float32)

    first = jnp.logical_and(init == 1, ic == 0)

    @pl.when(first)
    def _():
        out_ref[...] = contrib

    @pl.when(jnp.logical_not(first))
    def _():
        out_ref[...] += contrib


def kernel(hidden_states, gate_w, w1s, w2s, w3s):
    t, h_dim = hidden_states.shape
    n_exp, _, i_dim = w1s.shape
    top_k = 2
    n = t * top_k
    m = _M
    num_m = n // m
    n_ic = i_dim // _IC
    n_pairs = num_m + n_exp - 1

    x = hidden_states.reshape(-1, h_dim)

    weights, experts = pl.pallas_call(
        _router_body,
        out_shape=[
            jax.ShapeDtypeStruct((t, top_k), jnp.float32),
            jax.ShapeDtypeStruct((t, top_k), jnp.int32),
        ],
    )(x, gate_w)

    flat_sel = experts.reshape(-1)
    order = jnp.argsort(flat_sel, stable=True)
    token_idx = order // top_k
    x_sorted = jnp.take(x, token_idx, axis=0)
    rw_sorted = jnp.take(weights.reshape(-1), order, axis=0).reshape(n, 1)

    sizes = jnp.bincount(flat_sel, length=n_exp)
    offsets = jnp.concatenate([jnp.zeros((1,), jnp.int32),
                               jnp.cumsum(sizes).astype(jnp.int32)])
    tile_lo = (jnp.arange(num_m, dtype=jnp.int32) * m)[:, None]
    seg_lo = offsets[:-1][None, :]
    seg_hi = offsets[1:][None, :]
    ov_lo = jnp.maximum(seg_lo, tile_lo)
    ov_hi = jnp.minimum(seg_hi, tile_lo + m)
    active = ov_hi > ov_lo
    mm = jnp.broadcast_to(tile_lo // m, active.shape)
    ee = jnp.broadcast_to(jnp.arange(n_exp, dtype=jnp.int32)[None, :], active.shape)
    score = jnp.where(active, mm * n_exp + ee, num_m * n_exp + n_exp).reshape(-1)
    order64 = jnp.argsort(score)
    num_active = jnp.sum(active.astype(jnp.int32))
    sel = jnp.where(jnp.arange(n_pairs) < num_active,
                    order64[:n_pairs], order64[num_active - 1])
    m_p = (sel // n_exp).astype(jnp.int32)
    e_p = (sel % n_exp).astype(jnp.int32)
    lo_p = jnp.maximum(offsets[e_p], m_p * m) - m_p * m
    hi_p = jnp.minimum(offsets[e_p + 1], (m_p + 1) * m) - m_p * m
    hi_p = jnp.where(jnp.arange(n_pairs) < num_active, hi_p, lo_p)
    init_p = jnp.concatenate([jnp.ones((1,), jnp.int32),
                              (m_p[1:] != m_p[:-1]).astype(jnp.int32)])
    meta = jnp.stack([m_p, e_p, lo_p, hi_p, init_p], axis=1).astype(jnp.int32)

    grid_spec = pltpu.PrefetchScalarGridSpec(
        num_scalar_prefetch=1,
        grid=(n_pairs, n_ic),
        in_specs=[
            pl.BlockSpec((m, h_dim), lambda p, ic, md: (md[p, 0], 0)),
            pl.BlockSpec((m, 1), lambda p, ic, md: (md[p, 0], 0)),
            pl.BlockSpec((1, h_dim, _IC), lambda p, ic, md: (md[p, 1], 0, ic)),
            pl.BlockSpec((1, h_dim, _IC), lambda p, ic, md: (md[p, 1], 0, ic)),
            pl.BlockSpec((1, _IC, h_dim), lambda p, ic, md: (md[p, 1], ic, 0)),
        ],
        out_specs=pl.BlockSpec((m, h_dim), lambda p, ic, md: (md[p, 0], 0)),
    )
    sorted_out = pl.pallas_call(
        _mlp_body,
        grid_spec=grid_spec,
        out_shape=jax.ShapeDtypeStruct((n, h_dim), jnp.float32),
        compiler_params=pltpu.CompilerParams(
            dimension_semantics=("arbitrary", "arbitrary"),
        ),
    )(meta, x_sorted, rw_sorted, w1s, w3s, w2s)

    inv = jnp.argsort(order)
    merged = jnp.take(sorted_out, inv, axis=0).reshape(t, top_k, h_dim)
    return merged.sum(axis=1)
